# Initial kernel scaffold; baseline (speedup 1.0000x reference)
import jax
import jax.numpy as jnp
from jax.experimental import pallas as pl


def kernel(x, edge_index, edge_weight):
    raise NotImplementedError("write your pallas kernel here")


# speedup vs baseline: 5.3542x; 5.3542x over previous
import functools

import jax
import jax.numpy as jnp
from jax import lax
from jax.experimental import pallas as pl
from jax.experimental.pallas import tpu as pltpu
from jax.experimental.pallas import tpu_sc as plsc

_N = 10000
_E = 320000
_D = 128

_NC = 2
_NS = 16
_NW = _NC * _NS
_GROUP = 128
_NGROUPS = _E // _GROUP
_RPT = 632
_NPAD = _NS * _RPT


def _sc_spmm(x, row, col, w, zeros):
    mesh = plsc.VectorSubcoreMesh(core_axis_name="c", subcore_axis_name="s")

    @functools.partial(
        pl.kernel,
        mesh=mesh,
        out_type=jax.ShapeDtypeStruct((_NC, _NPAD, _D), jnp.float32),
        scratch_types=[
            pltpu.VMEM((_GROUP,), jnp.int32),
            pltpu.VMEM((_GROUP,), jnp.int32),
            pltpu.VMEM((_GROUP,), jnp.float32),
            pltpu.VMEM((_GROUP, _D), jnp.float32),
            pltpu.VMEM_SHARED((_NPAD, _D), jnp.float32),
            pltpu.SemaphoreType.DMA,
        ],
    )
    def k(x_hbm, row_hbm, col_hbm, w_hbm, z_hbm, out_hbm,
          col_v, row_v, w_v, rows_v, acc_sh, sem):
        cid = lax.axis_index("c")
        sid = lax.axis_index("s")
        wid = sid * _NC + cid

        pltpu.sync_copy(z_hbm, acc_sh.at[pl.ds(sid * _RPT, _RPT)])
        plsc.subcore_barrier()

        n_mine = (_NGROUPS - wid + _NW - 1) // _NW

        def group_body(t, carry):
            g = wid + t * _NW
            base = g * _GROUP
            pltpu.sync_copy(col_hbm.at[pl.ds(base, _GROUP)], col_v)
            pltpu.sync_copy(row_hbm.at[pl.ds(base, _GROUP)], row_v)
            pltpu.sync_copy(w_hbm.at[pl.ds(base, _GROUP)], w_v)
            pltpu.async_copy(x_hbm.at[col_v], rows_v, sem).wait()

            def escale(s, c2):
                wv16 = w_v[pl.ds(s * 16, 16)]
                for j in range(16):
                    e = s * 16 + j
                    wv = jnp.full((16,), wv16[j], dtype=jnp.float32)
                    for dd in range(_D // 16):
                        sl = pl.ds(dd * 16, 16)
                        rows_v[e, sl] = rows_v[e, sl] * wv
                return c2

            lax.fori_loop(0, _GROUP // 16, escale, 0)
            pltpu.sync_copy(rows_v, acc_sh.at[row_v], add=True)
            return carry

        lax.fori_loop(0, n_mine, group_body, 0)
        plsc.subcore_barrier()
        pltpu.sync_copy(acc_sh.at[pl.ds(sid * _RPT, _RPT)],
                        out_hbm.at[cid, pl.ds(sid * _RPT, _RPT)])

    return k(x, row, col, w, zeros)


def _add_body(a_ref, o_ref):
    o_ref[...] = a_ref[0] + a_ref[1]


def _combine(partials):
    blk = 1000
    return pl.pallas_call(
        _add_body,
        grid=(_N // blk,),
        in_specs=[pl.BlockSpec((_NC, blk, _D), lambda i: (0, i, 0))],
        out_specs=pl.BlockSpec((blk, _D), lambda i: (i, 0)),
        out_shape=jax.ShapeDtypeStruct((_N, _D), jnp.float32),
    )(partials)


def kernel(x, edge_index, edge_weight):
    row = edge_index[0]
    col = edge_index[1]
    zeros = jnp.zeros((_RPT, _D), jnp.float32)
    partials = _sc_spmm(x, row, col, edge_weight, zeros)
    return _combine(partials[:, :_N])
